# Initial kernel scaffold; baseline (speedup 1.0000x reference)
import jax
import jax.numpy as jnp
from jax.experimental import pallas as pl


def kernel(ori_feat, struc_feat, edge_index, W1o, b1o, W2o, b2o, W1s, b1s, W2s, b2s, M1, mb1, M2, mb2):
    raise NotImplementedError("write your pallas kernel here")


# speedup vs baseline: 5.6041x; 5.6041x over previous
import functools

import jax
import jax.numpy as jnp
from jax import lax
from jax.experimental import pallas as pl
from jax.experimental.pallas import tpu as pltpu
from jax.experimental.pallas import tpu_sc as plsc

N = 10000
E = 320000
NCORES = 2
NSUB = 16
NW = NCORES * NSUB
EPW = E // NW
BATCH = 80
NBATCH = EPW // BATCH
NPAD = 10240
ROWS_PT = NPAD // NSUB
DEGW = 8


def _sc_mesh():
    return plsc.VectorSubcoreMesh(core_axis_name="c", subcore_axis_name="s")


def _deg_body(src_hbm, dst_hbm, ones_hbm, zeros_hbm, od_out, id_out,
              src_v, dst_v, ones_v, od_acc, id_acc):
    c = lax.axis_index("c")
    s = lax.axis_index("s")
    wid = s * NCORES + c
    base = s * ROWS_PT
    pltpu.sync_copy(ones_hbm, ones_v)
    pltpu.sync_copy(zeros_hbm.at[pl.ds(base, ROWS_PT)], od_acc.at[pl.ds(base, ROWS_PT)])
    pltpu.sync_copy(zeros_hbm.at[pl.ds(base, ROWS_PT)], id_acc.at[pl.ds(base, ROWS_PT)])
    plsc.subcore_barrier()
    ebase = wid * EPW

    def step(j, carry):
        off = ebase + j * BATCH
        pltpu.sync_copy(src_hbm.at[pl.ds(off, BATCH)], src_v)
        pltpu.sync_copy(dst_hbm.at[pl.ds(off, BATCH)], dst_v)
        pltpu.sync_copy(ones_v, od_acc.at[src_v], add=True)
        pltpu.sync_copy(ones_v, id_acc.at[dst_v], add=True)
        return carry

    lax.fori_loop(0, NBATCH, step, 0)
    plsc.subcore_barrier()
    pltpu.sync_copy(od_acc.at[pl.ds(base, ROWS_PT)], od_out.at[c, pl.ds(base, ROWS_PT)])
    pltpu.sync_copy(id_acc.at[pl.ds(base, ROWS_PT)], id_out.at[c, pl.ds(base, ROWS_PT)])


def _degree_pass(src, dst, ones_deg, zeros_deg):
    f = pl.kernel(
        _deg_body,
        mesh=_sc_mesh(),
        compiler_params=pltpu.CompilerParams(use_tc_tiling_on_sc=False),
        out_type=[
            jax.ShapeDtypeStruct((NCORES, NPAD, DEGW), jnp.float32),
            jax.ShapeDtypeStruct((NCORES, NPAD, DEGW), jnp.float32),
        ],
        scratch_types=[
            pltpu.VMEM((BATCH,), jnp.int32),
            pltpu.VMEM((BATCH,), jnp.int32),
            pltpu.VMEM((BATCH, DEGW), jnp.float32),
            pltpu.VMEM_SHARED((NPAD, DEGW), jnp.float32),
            pltpu.VMEM_SHARED((NPAD, DEGW), jnp.float32),
        ],
    )
    return f(src, dst, ones_deg, zeros_deg)


def _make_agg_body(width):
    def body(x_hbm, src_hbm, dst_hbm, zeros_hbm, out_hbm,
             src_v, dst_v, rows_v, acc, sem):
        c = lax.axis_index("c")
        s = lax.axis_index("s")
        wid = s * NCORES + c
        base = s * ROWS_PT
        pltpu.sync_copy(zeros_hbm.at[pl.ds(base, ROWS_PT)], acc.at[pl.ds(base, ROWS_PT)])
        plsc.subcore_barrier()
        ebase = wid * EPW

        def step(j, carry):
            off = ebase + j * BATCH
            pltpu.sync_copy(src_hbm.at[pl.ds(off, BATCH)], src_v)
            pltpu.sync_copy(dst_hbm.at[pl.ds(off, BATCH)], dst_v)
            pltpu.async_copy(x_hbm.at[src_v], rows_v, sem).wait()
            pltpu.sync_copy(rows_v, acc.at[dst_v], add=True)
            return carry

        lax.fori_loop(0, NBATCH, step, 0)
        plsc.subcore_barrier()
        pltpu.sync_copy(acc.at[pl.ds(base, ROWS_PT)], out_hbm.at[c, pl.ds(base, ROWS_PT)])

    return body


def _agg_pass(x, src, dst, zeros, width):
    f = pl.kernel(
        _make_agg_body(width),
        mesh=_sc_mesh(),
        compiler_params=pltpu.CompilerParams(use_tc_tiling_on_sc=False),
        out_type=jax.ShapeDtypeStruct((NCORES, NPAD, width), jnp.float32),
        scratch_types=[
            pltpu.VMEM((BATCH,), jnp.int32),
            pltpu.VMEM((BATCH,), jnp.int32),
            pltpu.VMEM((BATCH, width), jnp.float32),
            pltpu.VMEM_SHARED((NPAD, width), jnp.float32),
            pltpu.SemaphoreType.DMA,
        ],
    )
    return f(x, src, dst, zeros)


def _tc_u1_body(ori_ref, struc_ref, w1o_ref, w1s_ref, out_ref):
    o = jnp.dot(ori_ref[...], w1o_ref[...], preferred_element_type=jnp.float32)
    st = jnp.dot(struc_ref[...], w1s_ref[...], preferred_element_type=jnp.float32)
    out_ref[...] = jnp.concatenate([o, st], axis=1)


def _tc_scale_body(u1_ref, odp_ref, out_ref):
    odp = odp_ref[...]
    od = jnp.maximum(odp[0, :N, 0:1] + odp[1, :N, 0:1], 1.0)
    out_ref[...] = u1_ref[...] * lax.rsqrt(od)


def _tc_mid_body(aggp_ref, odp_ref, idp_ref, b1o_ref, b1s_ref, w2o_ref, w2s_ref, out_ref):
    odp = odp_ref[...]
    idp = idp_ref[...]
    odn = lax.rsqrt(jnp.maximum(odp[0, :N, 0:1] + odp[1, :N, 0:1], 1.0))
    idn = lax.rsqrt(jnp.maximum(idp[0, :N, 0:1] + idp[1, :N, 0:1], 1.0))
    aggp = aggp_ref[...]
    agg = (aggp[0, :N] + aggp[1, :N]) * idn
    h1o = jax.nn.relu(agg[:, :64] + b1o_ref[...]) * odn
    h1s = jax.nn.relu(agg[:, 64:] + b1s_ref[...]) * odn
    x2o = jnp.dot(h1o, w2o_ref[...], preferred_element_type=jnp.float32)
    x2s = jnp.dot(h1s, w2s_ref[...], preferred_element_type=jnp.float32)
    out_ref[...] = jnp.concatenate([x2o, x2s], axis=1)


def _tc_mlp_body(aggp_ref, idp_ref, bcat_ref, m1_ref, mb1_ref, m2_ref, mb2_ref, out_ref):
    idp = idp_ref[...]
    idn = lax.rsqrt(jnp.maximum(idp[0, :N, 0:1] + idp[1, :N, 0:1], 1.0))
    aggp = aggp_ref[...]
    hc = (aggp[0, :N] + aggp[1, :N]) * idn + bcat_ref[...]
    h = jax.nn.relu(jnp.dot(hc, m1_ref[...], preferred_element_type=jnp.float32) + mb1_ref[...])
    out_ref[...] = jnp.dot(h, m2_ref[...], preferred_element_type=jnp.float32) + mb2_ref[...]


def _tc_call(body, out_shape, *args):
    return pl.pallas_call(body, out_shape=out_shape)(*args)


def kernel(ori_feat, struc_feat, edge_index, W1o, b1o, W2o, b2o,
           W1s, b1s, W2s, b2s, M1, mb1, M2, mb2):
    src = edge_index[0]
    dst = edge_index[1]

    ones_deg = jnp.ones((BATCH, DEGW), jnp.float32)
    zeros_deg = jnp.zeros((NPAD, DEGW), jnp.float32)
    zeros128 = jnp.zeros((NPAD, 128), jnp.float32)
    zeros32 = jnp.zeros((NPAD, 32), jnp.float32)

    odp, idp = _degree_pass(src, dst, ones_deg, zeros_deg)

    u1 = _tc_call(_tc_u1_body, jax.ShapeDtypeStruct((N, 128), jnp.float32),
                  ori_feat, struc_feat, W1o, W1s)
    x1 = _tc_call(_tc_scale_body, jax.ShapeDtypeStruct((N, 128), jnp.float32),
                  u1, odp)

    agg1 = _agg_pass(x1, src, dst, zeros128, 128)

    x2 = _tc_call(_tc_mid_body, jax.ShapeDtypeStruct((N, 32), jnp.float32),
                  agg1, odp, idp, b1o.reshape(1, 64), b1s.reshape(1, 64), W2o, W2s)

    agg2 = _agg_pass(x2, src, dst, zeros32, 32)

    bcat = jnp.concatenate([b2o, b2s]).reshape(1, 32)
    out = _tc_call(_tc_mlp_body, jax.ShapeDtypeStruct((N, 16), jnp.float32),
                   agg2, idp, bcat, M1, mb1.reshape(1, 64), M2, mb2.reshape(1, 16))
    return out
